# baseline (speedup 1.0000x reference)
import functools

import jax
import jax.numpy as jnp
from jax import lax
from jax.experimental import pallas as pl
from jax.experimental.pallas import tpu as pltpu
from jax.experimental.pallas import tpu_sc as plsc

N = 10000
E = 320000
G = 64
D_IN = 128
D_HID = 256

NC = 2
NS = 16
NW = NC * NS
RPT = 320
NPAD = NW * RPT
TRASH = RPT

SSL = 4096
NSL = 80
E_PAD = NSL * SSL
STB = 2048
STCAP = STB + 32
CAP = E_PAD + STB
NBLK = CAP // STB
GL = 64

BM = 1000
GRID = N // BM

_MESH = plsc.VectorSubcoreMesh(core_axis_name="c", subcore_axis_name="s")


def _wid():
    return lax.axis_index("c") * NS + lax.axis_index("s")


def _trash_fill(stage, lo_slot, n_slots):
    tr = jnp.full((16,), TRASH, jnp.int32)

    def body(i, carry):
        stage[pl.ds((lo_slot + i) * 16, 16)] = tr
        return carry

    lax.fori_loop(0, n_slots, body, 0)


def _sc_pre_body(src_hbm, dst_hbm, list_hbm, cnt_hbm,
                 src_st, dst_st, stage, tmp16):
    wid = _wid()
    lo = wid * RPT
    hi = lo + RPT

    _trash_fill(stage, 0, STCAP // 16)

    def slab_body(sl, carry):
        cnt, off = carry
        base = sl * SSL
        pltpu.sync_copy(src_hbm.at[pl.ds(base, SSL)], src_st)
        pltpu.sync_copy(dst_hbm.at[pl.ds(base, SSL)], dst_st)

        def group_body(g, carry):
            cnt, off = carry
            o = g * 16
            d_vec = dst_st[pl.ds(o, 16)]
            s_vec = src_st[pl.ds(o, 16)]
            inr = (d_vec >= lo) & (d_vec < hi)
            m = jnp.where(inr, 1, 0)
            packed = jnp.where(inr, s_vec * 512 + (d_vec - lo), TRASH)
            for l in range(16):
                stage[pl.ds(cnt, 16)] = jnp.full((16,), packed[l], jnp.int32)
                cnt = cnt + m[l]

            @pl.when(cnt >= STB)
            def _():
                pltpu.sync_copy(
                    stage.at[pl.ds(0, STB)],
                    list_hbm.at[pl.ds((wid * NBLK + off) * STB, STB)])
                rem = stage[pl.ds(STB, 16)]
                stage[pl.ds(0, 16)] = rem
                _trash_fill(stage, 1, STCAP // 16 - 1)

            off = jnp.where(cnt >= STB, off + 1, off)
            cnt = jnp.where(cnt >= STB, cnt - STB, cnt)
            return (cnt, off)

        return lax.fori_loop(0, SSL // 16, group_body, (cnt, off))

    cnt, off = lax.fori_loop(0, NSL, slab_body,
                             (jnp.int32(0), jnp.int32(0)))

    pltpu.sync_copy(stage.at[pl.ds(0, STB)],
                    list_hbm.at[pl.ds((wid * NBLK + off) * STB, STB)])
    tmp16[pl.ds(0, 16)] = jnp.full((16,), off + 1, jnp.int32)
    pltpu.sync_copy(tmp16, cnt_hbm.at[pl.ds(wid * 16, 16)])


_sc_pre = functools.partial(
    pl.kernel,
    mesh=_MESH,
    out_type=[
        jax.ShapeDtypeStruct((NW * CAP,), jnp.int32),
        jax.ShapeDtypeStruct((NW * 16,), jnp.int32),
    ],
    scratch_types=[
        pltpu.VMEM((SSL,), jnp.int32),
        pltpu.VMEM((SSL,), jnp.int32),
        pltpu.VMEM((STCAP,), jnp.int32),
        pltpu.VMEM((16,), jnp.int32),
    ],
)(_sc_pre_body)


def _zero_acc(acc, rows, width):
    per = width // 16
    z = jnp.zeros((16,), jnp.float32)

    def body(i, carry):
        acc[i // per, pl.ds((i % per) * 16, 16)] = z
        return carry

    lax.fori_loop(0, rows * per, body, 0)


def _read_nblk(cnt_hbm, wid, tmp16):
    pltpu.sync_copy(cnt_hbm.at[pl.ds(wid * 16, 16)], tmp16)
    return tmp16[pl.ds(0, 16)][0]


def _sc_deg_body(list_hbm, cnt_hbm, out_hbm, list_v, tmp16, acc):
    wid = _wid()
    nblk = _read_nblk(cnt_hbm, wid, tmp16)
    _zero_acc(acc, RPT + 8, 16)
    ones = jnp.ones((16,), jnp.float32)

    def blk_body(s, carry):
        pltpu.sync_copy(list_hbm.at[pl.ds((wid * NBLK + s) * STB, STB)],
                        list_v)

        def grp_body(g, carry2):
            pk = list_v[pl.ds(g * 16, 16)]
            dl = jnp.bitwise_and(pk, 511)
            for l in range(16):
                plsc.addupdate(acc.at[dl[l], pl.ds(0, 16)], ones)
            return carry2

        return lax.fori_loop(0, STB // 16, grp_body, carry)

    lax.fori_loop(0, nblk, blk_body, 0)
    pltpu.sync_copy(acc.at[pl.ds(0, RPT)],
                    out_hbm.at[pl.ds(wid * RPT, RPT)])


_sc_deg = functools.partial(
    pl.kernel,
    mesh=_MESH,
    out_type=jax.ShapeDtypeStruct((NPAD, 16), jnp.float32),
    scratch_types=[
        pltpu.VMEM((STB,), jnp.int32),
        pltpu.VMEM((16,), jnp.int32),
        pltpu.VMEM((RPT + 8, 16), jnp.float32),
    ],
)(_sc_deg_body)


def _sc_scatter_body(p_hbm, list_hbm, cnt_hbm, out_hbm,
                     list_v, sidx_v, didx_v, tmp16, rows_v, acc, sem):
    wid = _wid()
    nblk = _read_nblk(cnt_hbm, wid, tmp16)
    _zero_acc(acc, RPT + 8, D_HID)

    def blk_body(s, carry):
        pltpu.sync_copy(list_hbm.at[pl.ds((wid * NBLK + s) * STB, STB)],
                        list_v)

        def chunk_body(g, carry2):
            o = g * GL
            for q in range(GL // 16):
                pk = list_v[pl.ds(o + q * 16, 16)]
                sidx_v[pl.ds(q * 16, 16)] = jnp.right_shift(pk, 9)
                didx_v[pl.ds(q * 16, 16)] = jnp.bitwise_and(pk, 511)
            pltpu.async_copy(p_hbm.at[sidx_v], rows_v, sem).wait()
            for q in range(GL // 16):
                dl = didx_v[pl.ds(q * 16, 16)]
                for l in range(16):
                    j = q * 16 + l
                    d = dl[l]
                    for m in range(D_HID // 16):
                        plsc.addupdate(acc.at[d, pl.ds(m * 16, 16)],
                                       rows_v[j, pl.ds(m * 16, 16)])
            return carry2

        return lax.fori_loop(0, STB // GL, chunk_body, carry)

    lax.fori_loop(0, nblk, blk_body, 0)
    pltpu.sync_copy(acc.at[pl.ds(0, RPT)],
                    out_hbm.at[pl.ds(wid * RPT, RPT)])


_sc_scatter = functools.partial(
    pl.kernel,
    mesh=_MESH,
    out_type=jax.ShapeDtypeStruct((NPAD, D_HID), jnp.float32),
    scratch_types=[
        pltpu.VMEM((STB,), jnp.int32),
        pltpu.VMEM((GL,), jnp.int32),
        pltpu.VMEM((GL,), jnp.int32),
        pltpu.VMEM((16,), jnp.int32),
        pltpu.VMEM((GL, D_HID), jnp.float32),
        pltpu.VMEM((RPT + 8, D_HID), jnp.float32),
        pltpu.SemaphoreType.DMA,
    ],
)(_sc_scatter_body)


def _m0_body(x_b, w_b, deg_b, p_b, dinv_b):
    di = lax.rsqrt(deg_b[...] + 1.0)
    p_b[...] = jnp.dot(x_b[...], w_b[...],
                       preferred_element_type=jnp.float32) * di
    dinv_b[...] = di


def _tc_first(x, W0, deg):
    return pl.pallas_call(
        _m0_body,
        grid=(GRID,),
        in_specs=[
            pl.BlockSpec((BM, D_IN), lambda i: (i, 0)),
            pl.BlockSpec((D_IN, D_HID), lambda i: (0, 0)),
            pl.BlockSpec((BM, 1), lambda i: (i, 0)),
        ],
        out_specs=[
            pl.BlockSpec((BM, D_HID), lambda i: (i, 0)),
            pl.BlockSpec((BM, 1), lambda i: (i, 0)),
        ],
        out_shape=[
            jax.ShapeDtypeStruct((N, D_HID), jnp.float32),
            jax.ShapeDtypeStruct((N, 1), jnp.float32),
        ],
    )(x, W0, deg)


def _mid_body(s_b, p_b, dinv_b, b_b, w_b, pn_b):
    a = dinv_b[...] * (s_b[...] + p_b[...]) + b_b[...]
    h = jnp.where(a > 0, a, jnp.exp(jnp.minimum(a, 0.0)) - 1.0)
    pn_b[...] = jnp.dot(h, w_b[...],
                        preferred_element_type=jnp.float32) * dinv_b[...]


def _tc_mid(s, p, dinv, b, W):
    return pl.pallas_call(
        _mid_body,
        grid=(GRID,),
        in_specs=[
            pl.BlockSpec((BM, D_HID), lambda i: (i, 0)),
            pl.BlockSpec((BM, D_HID), lambda i: (i, 0)),
            pl.BlockSpec((BM, 1), lambda i: (i, 0)),
            pl.BlockSpec((1, D_HID), lambda i: (0, 0)),
            pl.BlockSpec((D_HID, D_HID), lambda i: (0, 0)),
        ],
        out_specs=pl.BlockSpec((BM, D_HID), lambda i: (i, 0)),
        out_shape=jax.ShapeDtypeStruct((N, D_HID), jnp.float32),
    )(s, p, dinv, b.reshape(1, D_HID), W)


def _readout_body(s_b, p_b, dinv_b, b_b, batch_b, wh_b, bh_b, out_b, acc):
    i = pl.program_id(0)

    @pl.when(i == 0)
    def _():
        acc[...] = jnp.zeros((G, D_HID), jnp.float32)

    a = dinv_b[...] * (s_b[...] + p_b[...]) + b_b[...]
    h = jnp.where(a > 0, a, jnp.exp(jnp.minimum(a, 0.0)) - 1.0)
    seg = lax.broadcasted_iota(jnp.int32, (1, G), 1)
    onehot = (batch_b[...] == seg).astype(jnp.float32)
    acc[...] += lax.dot_general(onehot, h, (((0,), (0,)), ((), ())),
                                preferred_element_type=jnp.float32)

    @pl.when(i == GRID - 1)
    def _():
        out_b[...] = jnp.dot(acc[...], wh_b[...],
                             preferred_element_type=jnp.float32) + bh_b[...]


def _tc_readout(s, p, dinv, b, batch, Wh, bh):
    return pl.pallas_call(
        _readout_body,
        grid=(GRID,),
        in_specs=[
            pl.BlockSpec((BM, D_HID), lambda i: (i, 0)),
            pl.BlockSpec((BM, D_HID), lambda i: (i, 0)),
            pl.BlockSpec((BM, 1), lambda i: (i, 0)),
            pl.BlockSpec((1, D_HID), lambda i: (0, 0)),
            pl.BlockSpec((BM, 1), lambda i: (i, 0)),
            pl.BlockSpec((D_HID, 1), lambda i: (0, 0)),
            pl.BlockSpec((1, 1), lambda i: (0, 0)),
        ],
        out_specs=pl.BlockSpec((G, 1), lambda i: (0, 0)),
        out_shape=jax.ShapeDtypeStruct((G, 1), jnp.float32),
        scratch_shapes=[pltpu.VMEM((G, D_HID), jnp.float32)],
    )(s, p, dinv, b.reshape(1, D_HID), batch, Wh, bh.reshape(1, 1))


def kernel(x, edge_index, batch, W0, b0, W1, b1, W2, b2, W3, b3, Wh, bh):
    pad = E_PAD - E
    src = jnp.concatenate([edge_index[0], jnp.zeros((pad,), jnp.int32)])
    dst = jnp.concatenate([edge_index[1], jnp.full((pad,), N, jnp.int32)])

    lists, counts = _sc_pre(src, dst)
    deg = _sc_deg(lists, counts)[:N, :1]
    p0, dinv = _tc_first(x, W0, deg)

    def prop(p):
        return _sc_scatter(p, lists, counts)[:N]

    p1 = _tc_mid(prop(p0), p0, dinv, b0, W1)
    p2 = _tc_mid(prop(p1), p1, dinv, b1, W2)
    p3 = _tc_mid(prop(p2), p2, dinv, b2, W3)
    s4 = prop(p3)

    return _tc_readout(s4, p3, dinv, b3, batch[:, None], Wh, bh)


# speedup vs baseline: 1.4990x; 1.4990x over previous
import functools

import jax
import jax.numpy as jnp
from jax import lax
from jax.experimental import pallas as pl
from jax.experimental.pallas import tpu as pltpu
from jax.experimental.pallas import tpu_sc as plsc

N = 10000
E = 320000
G = 64
D_IN = 128
D_HID = 256

NC = 2
NS = 16
NW = NC * NS
RPT = 320
NPAD = NW * RPT
TRASH = RPT

SSL = 4096
NSL = 80
E_PAD = NSL * SSL
STB = 2048
STCAP = STB + 32
CAP = E_PAD + STB
NBLK = CAP // STB
FL = 64
NCHK = CAP // FL

BM = 1000
GRID = N // BM

_MESH = plsc.VectorSubcoreMesh(core_axis_name="c", subcore_axis_name="s")


def _wid():
    return lax.axis_index("c") * NS + lax.axis_index("s")


def _trash_fill(stage, lo_slot, n_slots):
    tr = jnp.full((16,), TRASH, jnp.int32)

    def body(i, carry):
        stage[pl.ds((lo_slot + i) * 16, 16)] = tr
        return carry

    lax.fori_loop(0, n_slots, body, 0)


def _sc_pre_body(src_hbm, dst_hbm, list_hbm, cnt_hbm,
                 src_st, dst_st, stage, tmp16):
    wid = _wid()
    lo = wid * RPT
    hi = lo + RPT

    _trash_fill(stage, 0, STCAP // 16)

    def slab_body(sl, carry):
        cnt, off = carry
        base = sl * SSL
        pltpu.sync_copy(src_hbm.at[pl.ds(base, SSL)], src_st)
        pltpu.sync_copy(dst_hbm.at[pl.ds(base, SSL)], dst_st)

        def group_body(g, carry):
            cnt, off = carry
            o = g * 16
            d_vec = dst_st[pl.ds(o, 16)]
            s_vec = src_st[pl.ds(o, 16)]
            inr = (d_vec >= lo) & (d_vec < hi)
            m = jnp.where(inr, 1, 0)
            packed = jnp.where(inr, s_vec * 512 + (d_vec - lo), TRASH)
            for l in range(16):
                stage[pl.ds(cnt, 16)] = jnp.full((16,), packed[l], jnp.int32)
                cnt = cnt + m[l]

            @pl.when(cnt >= STB)
            def _():
                pltpu.sync_copy(
                    stage.at[pl.ds(0, STB)],
                    list_hbm.at[pl.ds((wid * NBLK + off) * STB, STB)])
                rem = stage[pl.ds(STB, 16)]
                stage[pl.ds(0, 16)] = rem
                _trash_fill(stage, 1, STCAP // 16 - 1)

            off = jnp.where(cnt >= STB, off + 1, off)
            cnt = jnp.where(cnt >= STB, cnt - STB, cnt)
            return (cnt, off)

        return lax.fori_loop(0, SSL // 16, group_body, (cnt, off))

    cnt, off = lax.fori_loop(0, NSL, slab_body,
                             (jnp.int32(0), jnp.int32(0)))

    pltpu.sync_copy(stage.at[pl.ds(0, STB)],
                    list_hbm.at[pl.ds((wid * NBLK + off) * STB, STB)])
    n64 = (off * STB + cnt + FL - 1) // FL
    tmp16[pl.ds(0, 16)] = jnp.full((16,), n64, jnp.int32)
    pltpu.sync_copy(tmp16, cnt_hbm.at[pl.ds(wid * 16, 16)])


_sc_pre = functools.partial(
    pl.kernel,
    mesh=_MESH,
    out_type=[
        jax.ShapeDtypeStruct((NW * CAP,), jnp.int32),
        jax.ShapeDtypeStruct((NW * 16,), jnp.int32),
    ],
    scratch_types=[
        pltpu.VMEM((SSL,), jnp.int32),
        pltpu.VMEM((SSL,), jnp.int32),
        pltpu.VMEM((STCAP,), jnp.int32),
        pltpu.VMEM((16,), jnp.int32),
    ],
)(_sc_pre_body)


def _zero_acc(acc, rows, width):
    per = width // 16
    z = jnp.zeros((16,), jnp.float32)

    def body(i, carry):
        acc[i // per, pl.ds((i % per) * 16, 16)] = z
        return carry

    lax.fori_loop(0, rows * per, body, 0)


def _read_n64(cnt_hbm, wid, tmp16):
    pltpu.sync_copy(cnt_hbm.at[pl.ds(wid * 16, 16)], tmp16)
    return tmp16[pl.ds(0, 16)][0]


def _sc_deg_body(list_hbm, cnt_hbm, out_hbm, chunk_v, tmp16, acc):
    wid = _wid()
    n64 = _read_n64(cnt_hbm, wid, tmp16)
    _zero_acc(acc, RPT + 8, 16)
    ones = jnp.ones((16,), jnp.float32)

    def chunk_body(g, carry):
        pltpu.sync_copy(list_hbm.at[pl.ds((wid * NCHK + g) * FL, FL)],
                        chunk_v)
        for q in range(FL // 16):
            pk = chunk_v[pl.ds(q * 16, 16)]
            dl = jnp.bitwise_and(pk, 511)
            for l in range(16):
                plsc.addupdate(acc.at[dl[l], pl.ds(0, 16)], ones)
        return carry

    lax.fori_loop(0, n64, chunk_body, 0)
    pltpu.sync_copy(acc.at[pl.ds(0, RPT)],
                    out_hbm.at[pl.ds(wid * RPT, RPT)])


_sc_deg = functools.partial(
    pl.kernel,
    mesh=_MESH,
    out_type=jax.ShapeDtypeStruct((NPAD, 16), jnp.float32),
    scratch_types=[
        pltpu.VMEM((FL,), jnp.int32),
        pltpu.VMEM((16,), jnp.int32),
        pltpu.VMEM((RPT + 8, 16), jnp.float32),
    ],
)(_sc_deg_body)


def _sc_scatter_body(p_hbm, list_hbm, cnt_hbm, out_hbm,
                     chunk_v, sidx_v, didx_v, tmp16, rows_v, acc, sem):
    wid = _wid()
    n64 = _read_n64(cnt_hbm, wid, tmp16)
    _zero_acc(acc, RPT + 8, D_HID)

    def chunk_body(g, carry):
        pltpu.sync_copy(list_hbm.at[pl.ds((wid * NCHK + g) * FL, FL)],
                        chunk_v)
        for q in range(FL // 16):
            pk = chunk_v[pl.ds(q * 16, 16)]
            sidx_v[pl.ds(q * 16, 16)] = jnp.right_shift(pk, 9)
            didx_v[pl.ds(q * 16, 16)] = jnp.bitwise_and(pk, 511)
        pltpu.async_copy(p_hbm.at[sidx_v], rows_v, sem).wait()
        for q in range(FL // 16):
            dl = didx_v[pl.ds(q * 16, 16)]
            for l in range(16):
                j = q * 16 + l
                d = dl[l]
                for m in range(D_HID // 16):
                    plsc.addupdate(acc.at[d, pl.ds(m * 16, 16)],
                                   rows_v[j, pl.ds(m * 16, 16)])
        return carry

    lax.fori_loop(0, n64, chunk_body, 0)
    pltpu.sync_copy(acc.at[pl.ds(0, RPT)],
                    out_hbm.at[pl.ds(wid * RPT, RPT)])


_sc_scatter = functools.partial(
    pl.kernel,
    mesh=_MESH,
    out_type=jax.ShapeDtypeStruct((NPAD, D_HID), jnp.float32),
    scratch_types=[
        pltpu.VMEM((FL,), jnp.int32),
        pltpu.VMEM((FL,), jnp.int32),
        pltpu.VMEM((FL,), jnp.int32),
        pltpu.VMEM((16,), jnp.int32),
        pltpu.VMEM((FL, D_HID), jnp.float32),
        pltpu.VMEM((RPT + 8, D_HID), jnp.float32),
        pltpu.SemaphoreType.DMA,
    ],
)(_sc_scatter_body)


def _m0_body(x_b, w_b, deg_b, p_b, dinv_b):
    di = lax.rsqrt(deg_b[...] + 1.0)
    p_b[...] = jnp.dot(x_b[...], w_b[...],
                       preferred_element_type=jnp.float32) * di
    dinv_b[...] = di


def _tc_first(x, W0, deg):
    return pl.pallas_call(
        _m0_body,
        grid=(GRID,),
        in_specs=[
            pl.BlockSpec((BM, D_IN), lambda i: (i, 0)),
            pl.BlockSpec((D_IN, D_HID), lambda i: (0, 0)),
            pl.BlockSpec((BM, 1), lambda i: (i, 0)),
        ],
        out_specs=[
            pl.BlockSpec((BM, D_HID), lambda i: (i, 0)),
            pl.BlockSpec((BM, 1), lambda i: (i, 0)),
        ],
        out_shape=[
            jax.ShapeDtypeStruct((N, D_HID), jnp.float32),
            jax.ShapeDtypeStruct((N, 1), jnp.float32),
        ],
    )(x, W0, deg)


def _mid_body(s_b, p_b, dinv_b, b_b, w_b, pn_b):
    a = dinv_b[...] * (s_b[...] + p_b[...]) + b_b[...]
    h = jnp.where(a > 0, a, jnp.exp(jnp.minimum(a, 0.0)) - 1.0)
    pn_b[...] = jnp.dot(h, w_b[...],
                        preferred_element_type=jnp.float32) * dinv_b[...]


def _tc_mid(s, p, dinv, b, W):
    return pl.pallas_call(
        _mid_body,
        grid=(GRID,),
        in_specs=[
            pl.BlockSpec((BM, D_HID), lambda i: (i, 0)),
            pl.BlockSpec((BM, D_HID), lambda i: (i, 0)),
            pl.BlockSpec((BM, 1), lambda i: (i, 0)),
            pl.BlockSpec((1, D_HID), lambda i: (0, 0)),
            pl.BlockSpec((D_HID, D_HID), lambda i: (0, 0)),
        ],
        out_specs=pl.BlockSpec((BM, D_HID), lambda i: (i, 0)),
        out_shape=jax.ShapeDtypeStruct((N, D_HID), jnp.float32),
    )(s, p, dinv, b.reshape(1, D_HID), W)


def _readout_body(s_b, p_b, dinv_b, b_b, batch_b, wh_b, bh_b, out_b, acc):
    i = pl.program_id(0)

    @pl.when(i == 0)
    def _():
        acc[...] = jnp.zeros((G, D_HID), jnp.float32)

    a = dinv_b[...] * (s_b[...] + p_b[...]) + b_b[...]
    h = jnp.where(a > 0, a, jnp.exp(jnp.minimum(a, 0.0)) - 1.0)
    seg = lax.broadcasted_iota(jnp.int32, (1, G), 1)
    onehot = (batch_b[...] == seg).astype(jnp.float32)
    acc[...] += lax.dot_general(onehot, h, (((0,), (0,)), ((), ())),
                                preferred_element_type=jnp.float32)

    @pl.when(i == GRID - 1)
    def _():
        out_b[...] = jnp.dot(acc[...], wh_b[...],
                             preferred_element_type=jnp.float32) + bh_b[...]


def _tc_readout(s, p, dinv, b, batch, Wh, bh):
    return pl.pallas_call(
        _readout_body,
        grid=(GRID,),
        in_specs=[
            pl.BlockSpec((BM, D_HID), lambda i: (i, 0)),
            pl.BlockSpec((BM, D_HID), lambda i: (i, 0)),
            pl.BlockSpec((BM, 1), lambda i: (i, 0)),
            pl.BlockSpec((1, D_HID), lambda i: (0, 0)),
            pl.BlockSpec((BM, 1), lambda i: (i, 0)),
            pl.BlockSpec((D_HID, 1), lambda i: (0, 0)),
            pl.BlockSpec((1, 1), lambda i: (0, 0)),
        ],
        out_specs=pl.BlockSpec((G, 1), lambda i: (0, 0)),
        out_shape=jax.ShapeDtypeStruct((G, 1), jnp.float32),
        scratch_shapes=[pltpu.VMEM((G, D_HID), jnp.float32)],
    )(s, p, dinv, b.reshape(1, D_HID), batch, Wh, bh.reshape(1, 1))


def kernel(x, edge_index, batch, W0, b0, W1, b1, W2, b2, W3, b3, Wh, bh):
    pad = E_PAD - E
    src = jnp.concatenate([edge_index[0], jnp.zeros((pad,), jnp.int32)])
    dst = jnp.concatenate([edge_index[1], jnp.full((pad,), N, jnp.int32)])

    lists, counts = _sc_pre(src, dst)
    deg = _sc_deg(lists, counts)[:N, :1]
    p0, dinv = _tc_first(x, W0, deg)

    def prop(p):
        return _sc_scatter(p, lists, counts)[:N]

    p1 = _tc_mid(prop(p0), p0, dinv, b0, W1)
    p2 = _tc_mid(prop(p1), p1, dinv, b1, W2)
    p3 = _tc_mid(prop(p2), p2, dinv, b2, W3)
    s4 = prop(p3)

    return _tc_readout(s4, p3, dinv, b3, batch[:, None], Wh, bh)
